# baseline (speedup 1.0000x reference)
import jax
import jax.numpy as jnp
from jax.experimental import pallas as pl
from jax.experimental.pallas import tpu as pltpu

_NBUF = 6
_CHUNK = 256


def _silu_stream(x_hbm, o_hbm, in_bufs, out_bufs, in_sems, out_sems):
    i = pl.program_id(0)
    nsteps = pl.num_programs(0)
    slot = jax.lax.rem(i, _NBUF)

    def _in_copy(step, buf):
        return pltpu.make_async_copy(
            x_hbm.at[pl.ds(step * _CHUNK, _CHUNK)], in_bufs.at[buf], in_sems.at[buf]
        )

    def _out_copy(step, buf):
        return pltpu.make_async_copy(
            out_bufs.at[buf], o_hbm.at[pl.ds(step * _CHUNK, _CHUNK)], out_sems.at[buf]
        )

    @pl.when(i == 0)
    def _prologue():
        for b in range(_NBUF):
            _in_copy(b, b).start()

    _in_copy(i, slot).wait()

    @pl.when(i >= _NBUF)
    def _wait_prev_out():
        _out_copy(i, slot).wait()

    xv = in_bufs[slot]
    out_bufs[slot] = xv * jax.nn.sigmoid(xv)

    _out_copy(i, slot).start()

    @pl.when(i + _NBUF < nsteps)
    def _next_in():
        _in_copy(i + _NBUF, slot).start()

    @pl.when(i == nsteps - 1)
    def _epilogue():
        for b in range(_NBUF):
            _out_copy(0, b).wait()


def kernel(x, logit_decay, log_tau, log_beta, log_gamma):
    del logit_decay, log_tau, log_beta, log_gamma
    b, t, d = x.shape
    rows = b * t
    xf = x.reshape(rows, d)
    nsteps = rows // _CHUNK
    out = pl.pallas_call(
        _silu_stream,
        grid=(nsteps,),
        in_specs=[pl.BlockSpec(memory_space=pltpu.MemorySpace.HBM)],
        out_specs=pl.BlockSpec(memory_space=pltpu.MemorySpace.HBM),
        out_shape=jax.ShapeDtypeStruct((rows, d), x.dtype),
        scratch_shapes=[
            pltpu.VMEM((_NBUF, _CHUNK, d), x.dtype),
            pltpu.VMEM((_NBUF, _CHUNK, d), x.dtype),
            pltpu.SemaphoreType.DMA((_NBUF,)),
            pltpu.SemaphoreType.DMA((_NBUF,)),
        ],
    )(xf)
    return out.reshape(b, t, d)


import functools
from jax.experimental.pallas import tpu_sc as plsc
from jax import lax

_SC_WORDS = 65536


def _sc_probe_body(x_hbm, out_hbm, buf, obuf, sem):
    wid = lax.axis_index("s") * 2 + lax.axis_index("c")
    base = wid * _SC_WORDS
    pltpu.sync_copy(x_hbm.at[pl.ds(base, _SC_WORDS)], buf)
    v = buf[pl.ds(0, 16)]
    obuf[...] = v * (1.0 / (1.0 + jnp.exp(-v)))
    pltpu.sync_copy(obuf, out_hbm.at[wid])


def _sc_probe(x_flat):
    mesh = plsc.VectorSubcoreMesh(core_axis_name="c", subcore_axis_name="s")
    k = functools.partial(
        pl.kernel,
        out_type=jax.ShapeDtypeStruct((32, 16), jnp.float32),
        mesh=mesh,
        scratch_types=[
            pltpu.VMEM((_SC_WORDS,), jnp.float32),
            pltpu.VMEM((16,), jnp.float32),
            pltpu.SemaphoreType.DMA,
        ],
    )(_sc_probe_body)
    return k(x_flat)


_tc_kernel_impl = kernel


def kernel(x, logit_decay, log_tau, log_beta, log_gamma):
    tc_out = _tc_kernel_impl(x, logit_decay, log_tau, log_beta, log_gamma)
    sc_in = x.reshape(-1)[: 32 * _SC_WORDS]
    sc_out = _sc_probe(sc_in)
    patched = lax.dynamic_update_slice(
        tc_out.reshape(x.shape[0] * x.shape[1], x.shape[2]), sc_out, (0, 0)
    )
    return patched.reshape(x.shape)


# speedup vs baseline: 1.9060x; 1.9060x over previous
import jax
import jax.numpy as jnp
from jax.experimental import pallas as pl
from jax.experimental.pallas import tpu as pltpu

_NBUF = 12
_CHUNK = 256


def _silu_stream(x_hbm, o_hbm, in_bufs, out_bufs, in_sems, out_sems):
    i = pl.program_id(0)
    nsteps = pl.num_programs(0)
    slot = jax.lax.rem(i, _NBUF)

    def _in_copy(step, buf):
        return pltpu.make_async_copy(
            x_hbm.at[pl.ds(step * _CHUNK, _CHUNK)], in_bufs.at[buf], in_sems.at[buf]
        )

    def _out_copy(step, buf):
        return pltpu.make_async_copy(
            out_bufs.at[buf], o_hbm.at[pl.ds(step * _CHUNK, _CHUNK)], out_sems.at[buf]
        )

    @pl.when(i == 0)
    def _prologue():
        for b in range(_NBUF):
            _in_copy(b, b).start()

    _in_copy(i, slot).wait()

    @pl.when(i >= _NBUF)
    def _wait_prev_out():
        _out_copy(i, slot).wait()

    xv = in_bufs[slot]
    out_bufs[slot] = xv * jax.nn.sigmoid(xv)

    _out_copy(i, slot).start()

    @pl.when(i + _NBUF < nsteps)
    def _next_in():
        _in_copy(i + _NBUF, slot).start()

    @pl.when(i == nsteps - 1)
    def _epilogue():
        for b in range(_NBUF):
            _out_copy(0, b).wait()


def kernel(x, logit_decay, log_tau, log_beta, log_gamma):
    del logit_decay, log_tau, log_beta, log_gamma
    b, t, d = x.shape
    rows = b * t
    xf = x.reshape(rows, d)
    nsteps = rows // _CHUNK
    out = pl.pallas_call(
        _silu_stream,
        grid=(nsteps,),
        in_specs=[pl.BlockSpec(memory_space=pltpu.MemorySpace.HBM)],
        out_specs=pl.BlockSpec(memory_space=pltpu.MemorySpace.HBM),
        out_shape=jax.ShapeDtypeStruct((rows, d), x.dtype),
        scratch_shapes=[
            pltpu.VMEM((_NBUF, _CHUNK, d), x.dtype),
            pltpu.VMEM((_NBUF, _CHUNK, d), x.dtype),
            pltpu.SemaphoreType.DMA((_NBUF,)),
            pltpu.SemaphoreType.DMA((_NBUF,)),
        ],
    )(xf)
    return out.reshape(b, t, d)


# speedup vs baseline: 1.9302x; 1.0127x over previous
import jax
import jax.numpy as jnp
from jax.experimental import pallas as pl
from jax.experimental.pallas import tpu as pltpu

_NBUF = 4
_CHUNK = 512


def _silu_stream(x_hbm, o_hbm, in_bufs, out_bufs, in_sems, out_sems):
    i = pl.program_id(0)
    nsteps = pl.num_programs(0)
    slot = jax.lax.rem(i, _NBUF)

    def _in_copy(step, buf):
        return pltpu.make_async_copy(
            x_hbm.at[pl.ds(step * _CHUNK, _CHUNK)], in_bufs.at[buf], in_sems.at[buf]
        )

    def _out_copy(step, buf):
        return pltpu.make_async_copy(
            out_bufs.at[buf], o_hbm.at[pl.ds(step * _CHUNK, _CHUNK)], out_sems.at[buf]
        )

    @pl.when(i == 0)
    def _prologue():
        for b in range(_NBUF):
            _in_copy(b, b).start()

    _in_copy(i, slot).wait()

    @pl.when(i >= _NBUF)
    def _wait_prev_out():
        _out_copy(i, slot).wait()

    xv = in_bufs[slot]
    out_bufs[slot] = xv * jax.nn.sigmoid(xv)

    _out_copy(i, slot).start()

    @pl.when(i + _NBUF < nsteps)
    def _next_in():
        _in_copy(i + _NBUF, slot).start()

    @pl.when(i == nsteps - 1)
    def _epilogue():
        for b in range(_NBUF):
            _out_copy(0, b).wait()


def kernel(x, logit_decay, log_tau, log_beta, log_gamma):
    del logit_decay, log_tau, log_beta, log_gamma
    b, t, d = x.shape
    rows = b * t
    xf = x.reshape(rows, d)
    nsteps = rows // _CHUNK
    out = pl.pallas_call(
        _silu_stream,
        grid=(nsteps,),
        in_specs=[pl.BlockSpec(memory_space=pltpu.MemorySpace.HBM)],
        out_specs=pl.BlockSpec(memory_space=pltpu.MemorySpace.HBM),
        out_shape=jax.ShapeDtypeStruct((rows, d), x.dtype),
        scratch_shapes=[
            pltpu.VMEM((_NBUF, _CHUNK, d), x.dtype),
            pltpu.VMEM((_NBUF, _CHUNK, d), x.dtype),
            pltpu.SemaphoreType.DMA((_NBUF,)),
            pltpu.SemaphoreType.DMA((_NBUF,)),
        ],
    )(xf)
    return out.reshape(b, t, d)
